# baseline (speedup 1.0000x reference)
import functools

import jax
import jax.numpy as jnp
from jax import lax
from jax.experimental import pallas as pl
from jax.experimental.pallas import tpu as pltpu
from jax.experimental.pallas import tpu_sc as plsc

N = 10000
E = 160000
HID = 256
NCLS = 40
NEG = 0.2
LANES = 16
NWORK = 32
EPT = E // NWORK
CH = 128
NCH = EPT // CH
REM = EPT - NCH * CH
SLAB = 624
TAIL = N - 16 * SLAB
ZR = 48
BLK = 200
HIGH = lax.Precision.HIGHEST


def _dot(a, b):
    return jnp.dot(a, b, preferred_element_type=jnp.float32)


def _elu(x):
    return jnp.where(x > 0.0, x, jnp.exp(jnp.minimum(x, 0.0)) - 1.0)


def _proj_tail(ft, ma_ref, mb_ref, ftsl_ref, ea_ref, eb_ref, nsl, slw):
    for s in range(nsl):
        ftsl_ref[s] = ft[:, s * slw:(s + 1) * slw]
    ea_ref[...] = _dot(ft, ma_ref[...])
    eb_ref[...] = _dot(ft, mb_ref[...])


def _tc0(x, w0, ma, mb):
    def body(x_ref, w_ref, ma_ref, mb_ref, ftsl_ref, ea_ref, eb_ref):
        ft = _dot(x_ref[...], w_ref[...])
        _proj_tail(ft, ma_ref, mb_ref, ftsl_ref, ea_ref, eb_ref, 16, 128)

    return pl.pallas_call(
        body,
        grid=(N // BLK,),
        in_specs=[
            pl.BlockSpec((BLK, 256), lambda i: (i, 0)),
            pl.BlockSpec((256, 2048), lambda i: (0, 0)),
            pl.BlockSpec((2048, 128), lambda i: (0, 0)),
            pl.BlockSpec((2048, 128), lambda i: (0, 0)),
        ],
        out_specs=[
            pl.BlockSpec((16, BLK, 128), lambda i: (0, i, 0)),
            pl.BlockSpec((BLK, 128), lambda i: (i, 0)),
            pl.BlockSpec((BLK, 128), lambda i: (i, 0)),
        ],
        out_shape=[
            jax.ShapeDtypeStruct((16, N, 128), jnp.float32),
            jax.ShapeDtypeStruct((N, 128), jnp.float32),
            jax.ShapeDtypeStruct((N, 128), jnp.float32),
        ],
    )(x, w0, ma, mb)


def _combine(p_ref, dp_ref, b_ref):
    u = jnp.concatenate([p_ref[0, s] + p_ref[1, s] for s in range(16)], axis=1)
    d = dp_ref[0, :, :16] + dp_ref[1, :, :16]
    cols = []
    for h in range(8):
        dh = d[:, h:h + 1]
        dsafe = jnp.where(dh > 0.0, dh, 1.0)
        cols.append(u[:, h * 256:(h + 1) * 256] / dsafe)
    return jnp.concatenate(cols, axis=1) + b_ref[...]


def _tcd0(p, dp, b, w1, ma, mb):
    def body(p_ref, dp_ref, b_ref, w_ref, ma_ref, mb_ref,
             h_ref, ftsl_ref, ea_ref, eb_ref):
        hn = _elu(_combine(p_ref, dp_ref, b_ref))
        h_ref[...] = hn
        ft = _dot(hn, w_ref[...])
        _proj_tail(ft, ma_ref, mb_ref, ftsl_ref, ea_ref, eb_ref, 16, 128)

    return pl.pallas_call(
        body,
        grid=(N // BLK,),
        in_specs=[
            pl.BlockSpec((2, 16, BLK, 128), lambda i: (0, 0, i, 0)),
            pl.BlockSpec((2, BLK, 128), lambda i: (0, i, 0)),
            pl.BlockSpec((1, 2048), lambda i: (0, 0)),
            pl.BlockSpec((2048, 2048), lambda i: (0, 0)),
            pl.BlockSpec((2048, 128), lambda i: (0, 0)),
            pl.BlockSpec((2048, 128), lambda i: (0, 0)),
        ],
        out_specs=[
            pl.BlockSpec((BLK, 2048), lambda i: (i, 0)),
            pl.BlockSpec((16, BLK, 128), lambda i: (0, i, 0)),
            pl.BlockSpec((BLK, 128), lambda i: (i, 0)),
            pl.BlockSpec((BLK, 128), lambda i: (i, 0)),
        ],
        out_shape=[
            jax.ShapeDtypeStruct((N, 2048), jnp.float32),
            jax.ShapeDtypeStruct((16, N, 128), jnp.float32),
            jax.ShapeDtypeStruct((N, 128), jnp.float32),
            jax.ShapeDtypeStruct((N, 128), jnp.float32),
        ],
    )(p, dp, b, w1, ma, mb)


def _tcd1(p, dp, b, hprev, w2p, ma2, mb2, wres2):
    def body(p_ref, dp_ref, b_ref, hp_ref, w_ref, ma_ref, mb_ref, wr_ref,
             ftsl_ref, ea_ref, eb_ref, res_ref):
        hn = _elu(_combine(p_ref, dp_ref, b_ref) + hp_ref[...])
        ft = _dot(hn, w_ref[...])
        _proj_tail(ft, ma_ref, mb_ref, ftsl_ref, ea_ref, eb_ref, 1, 128)
        res_ref[...] = _dot(hn, wr_ref[...])

    return pl.pallas_call(
        body,
        grid=(N // BLK,),
        in_specs=[
            pl.BlockSpec((2, 16, BLK, 128), lambda i: (0, 0, i, 0)),
            pl.BlockSpec((2, BLK, 128), lambda i: (0, i, 0)),
            pl.BlockSpec((1, 2048), lambda i: (0, 0)),
            pl.BlockSpec((BLK, 2048), lambda i: (i, 0)),
            pl.BlockSpec((2048, 128), lambda i: (0, 0)),
            pl.BlockSpec((128, 128), lambda i: (0, 0)),
            pl.BlockSpec((128, 128), lambda i: (0, 0)),
            pl.BlockSpec((2048, 40), lambda i: (0, 0)),
        ],
        out_specs=[
            pl.BlockSpec((1, BLK, 128), lambda i: (0, i, 0)),
            pl.BlockSpec((BLK, 128), lambda i: (i, 0)),
            pl.BlockSpec((BLK, 128), lambda i: (i, 0)),
            pl.BlockSpec((BLK, 40), lambda i: (i, 0)),
        ],
        out_shape=[
            jax.ShapeDtypeStruct((1, N, 128), jnp.float32),
            jax.ShapeDtypeStruct((N, 128), jnp.float32),
            jax.ShapeDtypeStruct((N, 128), jnp.float32),
            jax.ShapeDtypeStruct((N, 40), jnp.float32),
        ],
    )(p, dp, b, hprev, w2p, ma2, mb2, wres2)


def _tcf(p, dp, res, b2):
    def body(p_ref, dp_ref, res_ref, b_ref, o_ref):
        u = p_ref[0, 0] + p_ref[1, 0]
        d = (dp_ref[0, :, :16] + dp_ref[1, :, :16])[:, 0:1]
        dsafe = jnp.where(d > 0.0, d, 1.0)
        o_ref[...] = u[:, :NCLS] / dsafe + res_ref[...] + b_ref[...]

    return pl.pallas_call(
        body,
        grid=(N // BLK,),
        in_specs=[
            pl.BlockSpec((2, 1, BLK, 128), lambda i: (0, 0, i, 0)),
            pl.BlockSpec((2, BLK, 128), lambda i: (0, i, 0)),
            pl.BlockSpec((BLK, 40), lambda i: (i, 0)),
            pl.BlockSpec((1, 40), lambda i: (0, 0)),
        ],
        out_specs=pl.BlockSpec((BLK, 40), lambda i: (i, 0)),
        out_shape=jax.ShapeDtypeStruct((N, 40), jnp.float32),
    )(p, dp, res, b2)


def _sc_mesh():
    return plsc.VectorSubcoreMesh(core_axis_name="c", subcore_axis_name="s")


def _sc_logits(ei0, ei1, ea, eb):
    @functools.partial(
        pl.kernel,
        out_type=(
            jax.ShapeDtypeStruct((E * 16,), jnp.float32),
            jax.ShapeDtypeStruct((NWORK * 16,), jnp.float32),
        ),
        mesh=_sc_mesh(),
        scratch_types=[
            pltpu.VMEM((NCH, CH), jnp.int32),
            pltpu.VMEM((NCH, CH), jnp.int32),
            pltpu.VMEM((REM,), jnp.int32),
            pltpu.VMEM((REM,), jnp.int32),
            pltpu.VMEM((CH, 128), jnp.float32),
            pltpu.VMEM((CH, 128), jnp.float32),
            pltpu.VMEM((CH, 128), jnp.float32),
            pltpu.VMEM((CH, 128), jnp.float32),
            pltpu.VMEM((REM, 128), jnp.float32),
            pltpu.VMEM((REM, 128), jnp.float32),
            pltpu.VMEM((CH * 16,), jnp.float32),
            pltpu.VMEM((CH * 16,), jnp.float32),
            pltpu.VMEM((REM * 16,), jnp.float32),
            pltpu.VMEM((16,), jnp.float32),
            pltpu.SemaphoreType.DMA,
            pltpu.SemaphoreType.DMA,
        ],
    )
    def k(ei0_ref, ei1_ref, ea_ref, eb_ref, e_out, mx_out,
          idx2s, idx2d, sidx8, didx8, srA, drA, srB, drB, sr8, dr8,
          ebA, ebB, eb8, mxb, gA, gB):
        wid = lax.axis_index("s") * 2 + lax.axis_index("c")
        base = wid * EPT
        off8 = base + NCH * CH

        def ldids(c, _):
            off = base + c * CH
            pltpu.sync_copy(ei0_ref.at[pl.ds(off, CH)], idx2s.at[c])
            pltpu.sync_copy(ei1_ref.at[pl.ds(off, CH)], idx2d.at[c])
            return 0

        lax.fori_loop(0, NCH, ldids, 0)
        pltpu.sync_copy(ei0_ref.at[pl.ds(off8, REM)], sidx8)
        pltpu.sync_copy(ei1_ref.at[pl.ds(off8, REM)], didx8)

        def g_start(c, sr, dr, sem):
            pltpu.async_copy(ea_ref.at[idx2s.at[c]], sr, sem)
            pltpu.async_copy(eb_ref.at[idx2d.at[c]], dr, sem)

        def g_wait(c, sr, dr, sem):
            pltpu.make_async_copy(ea_ref.at[idx2s.at[c]], sr, sem).wait()
            pltpu.make_async_copy(eb_ref.at[idx2d.at[c]], dr, sem).wait()

        def compute(sr, dr, ebuf, mx):
            def ej4(q, mxq):
                for u in range(4):
                    j = q * 4 + u
                    v = sr[j, pl.ds(0, 16)] + dr[j, pl.ds(0, 16)]
                    ev = jnp.where(v > 0.0, v, NEG * v)
                    ebuf[pl.ds(j * 16, 16)] = ev
                    mxq = jnp.maximum(mxq, ev)
                return mxq

            return lax.fori_loop(0, CH // 4, ej4, mx)

        g_start(0, srA, drA, gA)

        def pair(cc, mx):
            c0 = 2 * cc
            c1 = c0 + 1
            g_start(c1, srB, drB, gB)
            g_wait(c0, srA, drA, gA)
            mx = compute(srA, drA, ebA, mx)
            pltpu.sync_copy(ebA, e_out.at[pl.ds((base + c0 * CH) * 16,
                                                CH * 16)])
            g_start(c0 + 2, srA, drA, gA)
            g_wait(c1, srB, drB, gB)
            mx = compute(srB, drB, ebB, mx)
            pltpu.sync_copy(ebB, e_out.at[pl.ds((base + c1 * CH) * 16,
                                                CH * 16)])
            return mx

        mx = lax.fori_loop(0, (NCH - 1) // 2, pair,
                           jnp.full((16,), -jnp.inf, jnp.float32))
        g_wait(NCH - 1, srA, drA, gA)
        mx = compute(srA, drA, ebA, mx)
        pltpu.sync_copy(ebA, e_out.at[pl.ds((base + (NCH - 1) * CH) * 16,
                                            CH * 16)])

        pltpu.async_copy(ea_ref.at[sidx8], sr8, gA)
        pltpu.async_copy(eb_ref.at[didx8], dr8, gB)
        pltpu.make_async_copy(ea_ref.at[sidx8], sr8, gA).wait()
        pltpu.make_async_copy(eb_ref.at[didx8], dr8, gB).wait()
        for jj in range(REM):
            v = sr8[jj, pl.ds(0, 16)] + dr8[jj, pl.ds(0, 16)]
            ev = jnp.where(v > 0.0, v, NEG * v)
            eb8[pl.ds(jj * 16, 16)] = ev
            mx = jnp.maximum(mx, ev)
        pltpu.sync_copy(eb8, e_out.at[pl.ds(off8 * 16, REM * 16)])
        mxb[...] = mx
        pltpu.sync_copy(mxb, mx_out.at[pl.ds(wid * 16, 16)])

    return k(ei0, ei1, ea, eb)


def _sc_messages(ei0, ei1, e_hbm, mxs, ftsl, nsl):
    slw = 128
    nz = slw // LANES

    @functools.partial(
        pl.kernel,
        out_type=(
            jax.ShapeDtypeStruct((2, N, 128), jnp.float32),
            jax.ShapeDtypeStruct((2, nsl, N, slw), jnp.float32),
        ),
        mesh=_sc_mesh(),
        scratch_types=[
            pltpu.VMEM((NCH, CH), jnp.int32),
            pltpu.VMEM((NCH, CH), jnp.int32),
            pltpu.VMEM((REM,), jnp.int32),
            pltpu.VMEM((REM,), jnp.int32),
            pltpu.VMEM((CH * 16,), jnp.float32),
            pltpu.VMEM((CH * 16,), jnp.float32),
            pltpu.VMEM((REM * 16,), jnp.float32),
            pltpu.VMEM((CH, slw), jnp.float32),
            pltpu.VMEM((CH, slw), jnp.float32),
            pltpu.VMEM((REM, slw), jnp.float32),
            pltpu.VMEM((NWORK * 16,), jnp.float32),
            pltpu.VMEM_SHARED((N, slw), jnp.float32),
            pltpu.SemaphoreType.DMA,
            pltpu.SemaphoreType.DMA,
            pltpu.SemaphoreType.DMA,
            pltpu.SemaphoreType.DMA,
            pltpu.SemaphoreType.DMA,
            pltpu.SemaphoreType.DMA,
            pltpu.SemaphoreType.DMA,
        ],
    )
    def k(ei0_ref, ei1_ref, e_ref, mx_ref, ft_ref, dpart, mpart,
          idx2s, idx2d, sidx8, didx8, erA, erB, er8, rowsA, rowsB, rows8,
          mxacc, macc, gsA, gsB, esA, esB, s8, ssA, ssB):
        cid = lax.axis_index("c")
        sid = lax.axis_index("s")
        wid = sid * 2 + cid
        base = wid * EPT
        slab = sid * SLAB
        off8 = base + NCH * CH
        zv = jnp.zeros((16,), jnp.float32)
        lane16 = lax.iota(jnp.int32, 16)

        def ldids(c, _):
            off = base + c * CH
            pltpu.sync_copy(ei0_ref.at[pl.ds(off, CH)], idx2s.at[c])
            pltpu.sync_copy(ei1_ref.at[pl.ds(off, CH)], idx2d.at[c])
            return 0

        lax.fori_loop(0, NCH, ldids, 0)
        pltpu.sync_copy(ei0_ref.at[pl.ds(off8, REM)], sidx8)
        pltpu.sync_copy(ei1_ref.at[pl.ds(off8, REM)], didx8)

        pltpu.sync_copy(mx_ref, mxacc)

        def mred(j, g):
            return jnp.maximum(g, mxacc[pl.ds(j * 16, 16)])

        gmax = lax.fori_loop(0, NWORK, mred,
                             jnp.full((16,), -jnp.inf, jnp.float32))
        msk = lane16 < 8

        def mkzrows(buf):
            def zr(j, _):
                for kk in range(nz):
                    buf[j, pl.ds(kk * 16, 16)] = zv
                return 0
            return zr

        zrows = mkzrows(rowsA)
        zrowsB = mkzrows(rowsB)

        def zrows8(j, _):
            for kk in range(nz):
                rows8[j, pl.ds(kk * 16, 16)] = zv
            return 0

        def zero_acc():
            for q in range(4):
                pltpu.sync_copy(rowsA, macc.at[pl.ds(slab + q * CH, CH)])
            pltpu.sync_copy(rowsA.at[pl.ds(0, SLAB - 4 * CH)],
                            macc.at[pl.ds(slab + 4 * CH, SLAB - 4 * CH)])

            @pl.when(sid == 15)
            def _():
                pltpu.sync_copy(rowsA.at[pl.ds(0, TAIL)],
                                macc.at[pl.ds(16 * SLAB, TAIL)])

        def gath_start(s, c, buf, sem):
            pltpu.async_copy(ft_ref.at[s].at[idx2s.at[c]], buf, sem)

        def gath_wait(s, c, buf, sem):
            pltpu.make_async_copy(ft_ref.at[s].at[idx2s.at[c]], buf,
                                  sem).wait()

        def eload_start(c, buf, sem):
            pltpu.async_copy(
                e_ref.at[pl.ds((base + c * CH) * 16, CH * 16)], buf, sem)

        def eload_wait(c, buf, sem):
            pltpu.make_async_copy(
                e_ref.at[pl.ds((base + c * CH) * 16, CH * 16)], buf,
                sem).wait()


        lax.fori_loop(0, CH, zrows, 0)
        lax.fori_loop(0, CH, zrowsB, 0)
        lax.fori_loop(0, REM, zrows8, 0)
        zero_acc()
        plsc.subcore_barrier()

        def fill(er_buf, rows_buf):
            def ej(j, _2):
                ee = jnp.where(msk,
                               jnp.exp(er_buf[pl.ds(j * 16, 16)] - gmax),
                               0.0)
                rows_buf[j, pl.ds(0, 16)] = ee
                return 0

            lax.fori_loop(0, CH, ej, 0)

        eload_start(0, erA, esA)

        def dpair(cc, _):
            c0 = 2 * cc
            c1 = c0 + 1

            @pl.when(cc > 0)
            def _():
                pltpu.make_async_copy(
                    rowsB, macc.at[idx2d.at[c0 - 1]], ssB).wait()

            eload_start(c1, erB, esB)
            eload_wait(c0, erA, esA)
            fill(erA, rowsA)
            pltpu.async_copy(rowsA, macc.at[idx2d.at[c0]], ssA, add=True)
            eload_wait(c1, erB, esB)
            fill(erB, rowsB)
            pltpu.async_copy(rowsB, macc.at[idx2d.at[c1]], ssB, add=True)
            pltpu.make_async_copy(rowsA, macc.at[idx2d.at[c0]], ssA).wait()
            eload_start(c0 + 2, erA, esA)
            return 0

        lax.fori_loop(0, (NCH - 1) // 2, dpair, 0)
        pltpu.make_async_copy(rowsB, macc.at[idx2d.at[NCH - 2]],
                              ssB).wait()
        eload_wait(NCH - 1, erA, esA)
        fill(erA, rowsA)
        pltpu.sync_copy(rowsA, macc.at[idx2d.at[NCH - 1]], add=True)

        pltpu.sync_copy(e_ref.at[pl.ds(off8 * 16, REM * 16)], er8)

        def ej8(j, _2):
            ee = jnp.where(msk,
                           jnp.exp(er8[pl.ds(j * 16, 16)] - gmax), 0.0)
            rows8[j, pl.ds(0, 16)] = ee
            return 0

        lax.fori_loop(0, REM, ej8, 0)
        pltpu.sync_copy(rows8, macc.at[didx8], add=True)

        plsc.subcore_barrier()
        pltpu.sync_copy(macc.at[pl.ds(slab, SLAB)],
                        dpart.at[cid, pl.ds(slab, SLAB)])

        @pl.when(sid == 15)
        def _():
            pltpu.sync_copy(macc.at[pl.ds(16 * SLAB, TAIL)],
                            dpart.at[cid, pl.ds(16 * SLAB, TAIL)])

        def slice_body(s, _s):
            h = s // 2
            hspl = jnp.broadcast_to(h, (16,))

            lax.fori_loop(0, CH, zrows, 0)
            zero_acc()
            plsc.subcore_barrier()

            gath_start(s, 0, rowsA, gsA)
            eload_start(0, erA, esA)

            def compute(rows_buf, er_buf):
                def ej4(q, _):
                    for u in range(8):
                        j = q * 8 + u
                        ev = er_buf[pl.ds(j * 16, 16)]
                        ee = jnp.exp(ev - gmax)
                        m = ee.at[hspl].get(mode="promise_in_bounds")
                        for kk in range(nz):
                            sl = pl.ds(kk * 16, 16)
                            rows_buf[j, sl] = rows_buf[j, sl] * m
                    return 0

                lax.fori_loop(0, CH // 8, ej4, 0)

            def pair(cc, _):
                c0 = 2 * cc
                c1 = c0 + 1

                @pl.when(cc > 0)
                def _():
                    pltpu.make_async_copy(
                        rowsB, macc.at[idx2d.at[c0 - 1]], ssB).wait()

                gath_start(s, c1, rowsB, gsB)
                eload_start(c1, erB, esB)
                gath_wait(s, c0, rowsA, gsA)
                eload_wait(c0, erA, esA)
                compute(rowsA, erA)
                pltpu.async_copy(rowsA, macc.at[idx2d.at[c0]], ssA,
                                 add=True)
                gath_wait(s, c1, rowsB, gsB)
                eload_wait(c1, erB, esB)
                compute(rowsB, erB)
                pltpu.async_copy(rowsB, macc.at[idx2d.at[c1]], ssB,
                                 add=True)
                pltpu.make_async_copy(rowsA, macc.at[idx2d.at[c0]],
                                      ssA).wait()
                gath_start(s, c0 + 2, rowsA, gsA)
                eload_start(c0 + 2, erA, esA)
                return 0

            npair = (NCH - 1) // 2
            lax.fori_loop(0, npair, pair, 0)
            pltpu.make_async_copy(rowsB, macc.at[idx2d.at[NCH - 2]],
                                  ssB).wait()
            gath_wait(s, NCH - 1, rowsA, gsA)
            eload_wait(NCH - 1, erA, esA)
            compute(rowsA, erA)
            pltpu.sync_copy(rowsA, macc.at[idx2d.at[NCH - 1]], add=True)

            cp8 = pltpu.async_copy(ft_ref.at[s].at[sidx8], rows8, s8)
            pltpu.sync_copy(e_ref.at[pl.ds(off8 * 16, REM * 16)], er8)
            cp8.wait()
            for jj in range(REM):
                ev = er8[pl.ds(jj * 16, 16)]
                ee = jnp.exp(ev - gmax)
                m = ee.at[hspl].get(mode="promise_in_bounds")
                for kk in range(nz):
                    sl = pl.ds(kk * 16, 16)
                    rows8[jj, sl] = rows8[jj, sl] * m
            pltpu.sync_copy(rows8, macc.at[didx8], add=True)

            plsc.subcore_barrier()
            pltpu.sync_copy(macc.at[pl.ds(slab, SLAB)],
                            mpart.at[cid, s, pl.ds(slab, SLAB)])

            @pl.when(sid == 15)
            def _():
                pltpu.sync_copy(macc.at[pl.ds(16 * SLAB, TAIL)],
                                mpart.at[cid, s, pl.ds(16 * SLAB, TAIL)])

            return 0

        lax.fori_loop(0, nsl, slice_body, 0)

    return k(ei0, ei1, e_hbm, mxs, ftsl)


def _mk_ab(al, ar, nh, d, kp):
    eye = jnp.eye(nh, dtype=jnp.float32)
    bdl = (al[:, :, None] * eye[:, None, :]).reshape(nh * d, nh)
    bdr = (ar[:, :, None] * eye[:, None, :]).reshape(nh * d, nh)
    z = jnp.zeros((kp, 8), jnp.float32)
    left = z.at[:nh * d, :nh].set(bdl)
    right = z.at[:nh * d, :nh].set(bdr)
    pad = jnp.zeros((kp, 96), jnp.float32)
    return (jnp.concatenate([left, right, pad], axis=1),
            jnp.concatenate([right, left, pad], axis=1))


def kernel(inputs, edge_index, W0, al0, ar0, b0, W1, al1, ar1, b1,
           W2, al2, ar2, b2, Wres2):
    ei0 = edge_index[0].astype(jnp.int32)
    ei1 = edge_index[1].astype(jnp.int32)
    ma0, mb0 = _mk_ab(al0, ar0, 8, HID, 2048)
    ma1, mb1 = _mk_ab(al1, ar1, 8, HID, 2048)
    ma2, mb2 = _mk_ab(al2, ar2, 1, NCLS, 128)
    w2p = jnp.zeros((2048, 128), jnp.float32).at[:, :NCLS].set(W2)

    ftsl0, ea0, eb0 = _tc0(inputs, W0, ma0, mb0)
    e0, mx0 = _sc_logits(ei0, ei1, ea0, eb0)
    dp0, mp0 = _sc_messages(ei0, ei1, e0, mx0, ftsl0, 16)

    h1, ftsl1, ea1, eb1 = _tcd0(mp0, dp0, b0.reshape(1, 2048), W1, ma1, mb1)
    e1, mx1 = _sc_logits(ei0, ei1, ea1, eb1)
    dp1, mp1 = _sc_messages(ei0, ei1, e1, mx1, ftsl1, 16)

    ftsl2, ea2, eb2, res2 = _tcd1(mp1, dp1, b1.reshape(1, 2048), h1,
                                  w2p, ma2, mb2, Wres2)
    e2, mx2 = _sc_logits(ei0, ei1, ea2, eb2)
    dp2, mp2 = _sc_messages(ei0, ei1, e2, mx2, ftsl2, 1)

    return _tcf(mp2, dp2, res2, b2.reshape(1, NCLS))


# speedup vs baseline: 1.7172x; 1.7172x over previous
import functools

import jax
import jax.numpy as jnp
from jax import lax
from jax.experimental import pallas as pl
from jax.experimental.pallas import tpu as pltpu
from jax.experimental.pallas import tpu_sc as plsc

N = 10000
E = 160000
HID = 256
NCLS = 40
NEG = 0.2
LANES = 16
NWORK = 32
EPT = E // NWORK
CH = 128
NCH = EPT // CH
REM = EPT - NCH * CH
SLAB = 624
TAIL = N - 16 * SLAB
ZR = 48
BLK = 200
HIGH = lax.Precision.HIGHEST


def _dot(a, b):
    return jnp.dot(a, b, preferred_element_type=jnp.float32)


def _elu(x):
    return jnp.where(x > 0.0, x, jnp.exp(jnp.minimum(x, 0.0)) - 1.0)


def _proj_tail(ft, ma_ref, mb_ref, ftsl_ref, ea_ref, eb_ref, nsl, slw):
    for s in range(nsl):
        ftsl_ref[s] = ft[:, s * slw:(s + 1) * slw]
    ea_ref[...] = _dot(ft, ma_ref[...])
    eb_ref[...] = _dot(ft, mb_ref[...])


def _tc0(x, w0, ma, mb):
    def body(x_ref, w_ref, ma_ref, mb_ref, ftsl_ref, ea_ref, eb_ref):
        ft = _dot(x_ref[...], w_ref[...])
        _proj_tail(ft, ma_ref, mb_ref, ftsl_ref, ea_ref, eb_ref, 16, 128)

    return pl.pallas_call(
        body,
        grid=(N // BLK,),
        in_specs=[
            pl.BlockSpec((BLK, 256), lambda i: (i, 0)),
            pl.BlockSpec((256, 2048), lambda i: (0, 0)),
            pl.BlockSpec((2048, 128), lambda i: (0, 0)),
            pl.BlockSpec((2048, 128), lambda i: (0, 0)),
        ],
        out_specs=[
            pl.BlockSpec((16, BLK, 128), lambda i: (0, i, 0)),
            pl.BlockSpec((BLK, 128), lambda i: (i, 0)),
            pl.BlockSpec((BLK, 128), lambda i: (i, 0)),
        ],
        out_shape=[
            jax.ShapeDtypeStruct((16, N, 128), jnp.float32),
            jax.ShapeDtypeStruct((N, 128), jnp.float32),
            jax.ShapeDtypeStruct((N, 128), jnp.float32),
        ],
    )(x, w0, ma, mb)


def _combine(p_ref, dp_ref, b_ref):
    u = jnp.concatenate([p_ref[0, s] + p_ref[1, s] for s in range(16)], axis=1)
    d = dp_ref[0, :, :16] + dp_ref[1, :, :16]
    cols = []
    for h in range(8):
        dh = d[:, h:h + 1]
        dsafe = jnp.where(dh > 0.0, dh, 1.0)
        cols.append(u[:, h * 256:(h + 1) * 256] / dsafe)
    return jnp.concatenate(cols, axis=1) + b_ref[...]


def _tcd0(p, dp, b, w1, ma, mb):
    def body(p_ref, dp_ref, b_ref, w_ref, ma_ref, mb_ref,
             h_ref, ftsl_ref, ea_ref, eb_ref):
        hn = _elu(_combine(p_ref, dp_ref, b_ref))
        h_ref[...] = hn
        ft = _dot(hn, w_ref[...])
        _proj_tail(ft, ma_ref, mb_ref, ftsl_ref, ea_ref, eb_ref, 16, 128)

    return pl.pallas_call(
        body,
        grid=(N // BLK,),
        in_specs=[
            pl.BlockSpec((2, 16, BLK, 128), lambda i: (0, 0, i, 0)),
            pl.BlockSpec((2, BLK, 128), lambda i: (0, i, 0)),
            pl.BlockSpec((1, 2048), lambda i: (0, 0)),
            pl.BlockSpec((2048, 2048), lambda i: (0, 0)),
            pl.BlockSpec((2048, 128), lambda i: (0, 0)),
            pl.BlockSpec((2048, 128), lambda i: (0, 0)),
        ],
        out_specs=[
            pl.BlockSpec((BLK, 2048), lambda i: (i, 0)),
            pl.BlockSpec((16, BLK, 128), lambda i: (0, i, 0)),
            pl.BlockSpec((BLK, 128), lambda i: (i, 0)),
            pl.BlockSpec((BLK, 128), lambda i: (i, 0)),
        ],
        out_shape=[
            jax.ShapeDtypeStruct((N, 2048), jnp.float32),
            jax.ShapeDtypeStruct((16, N, 128), jnp.float32),
            jax.ShapeDtypeStruct((N, 128), jnp.float32),
            jax.ShapeDtypeStruct((N, 128), jnp.float32),
        ],
    )(p, dp, b, w1, ma, mb)


def _tcd1(p, dp, b, hprev, w2p, ma2, mb2, wres2):
    def body(p_ref, dp_ref, b_ref, hp_ref, w_ref, ma_ref, mb_ref, wr_ref,
             ftsl_ref, ea_ref, eb_ref, res_ref):
        hn = _elu(_combine(p_ref, dp_ref, b_ref) + hp_ref[...])
        ft = _dot(hn, w_ref[...])
        _proj_tail(ft, ma_ref, mb_ref, ftsl_ref, ea_ref, eb_ref, 1, 128)
        res_ref[...] = _dot(hn, wr_ref[...])

    return pl.pallas_call(
        body,
        grid=(N // BLK,),
        in_specs=[
            pl.BlockSpec((2, 16, BLK, 128), lambda i: (0, 0, i, 0)),
            pl.BlockSpec((2, BLK, 128), lambda i: (0, i, 0)),
            pl.BlockSpec((1, 2048), lambda i: (0, 0)),
            pl.BlockSpec((BLK, 2048), lambda i: (i, 0)),
            pl.BlockSpec((2048, 128), lambda i: (0, 0)),
            pl.BlockSpec((128, 128), lambda i: (0, 0)),
            pl.BlockSpec((128, 128), lambda i: (0, 0)),
            pl.BlockSpec((2048, 40), lambda i: (0, 0)),
        ],
        out_specs=[
            pl.BlockSpec((1, BLK, 128), lambda i: (0, i, 0)),
            pl.BlockSpec((BLK, 128), lambda i: (i, 0)),
            pl.BlockSpec((BLK, 128), lambda i: (i, 0)),
            pl.BlockSpec((BLK, 40), lambda i: (i, 0)),
        ],
        out_shape=[
            jax.ShapeDtypeStruct((1, N, 128), jnp.float32),
            jax.ShapeDtypeStruct((N, 128), jnp.float32),
            jax.ShapeDtypeStruct((N, 128), jnp.float32),
            jax.ShapeDtypeStruct((N, 40), jnp.float32),
        ],
    )(p, dp, b, hprev, w2p, ma2, mb2, wres2)


def _tcf(p, dp, res, b2):
    def body(p_ref, dp_ref, res_ref, b_ref, o_ref):
        u = p_ref[0, 0] + p_ref[1, 0]
        d = (dp_ref[0, :, :16] + dp_ref[1, :, :16])[:, 0:1]
        dsafe = jnp.where(d > 0.0, d, 1.0)
        o_ref[...] = u[:, :NCLS] / dsafe + res_ref[...] + b_ref[...]

    return pl.pallas_call(
        body,
        grid=(N // BLK,),
        in_specs=[
            pl.BlockSpec((2, 1, BLK, 128), lambda i: (0, 0, i, 0)),
            pl.BlockSpec((2, BLK, 128), lambda i: (0, i, 0)),
            pl.BlockSpec((BLK, 40), lambda i: (i, 0)),
            pl.BlockSpec((1, 40), lambda i: (0, 0)),
        ],
        out_specs=pl.BlockSpec((BLK, 40), lambda i: (i, 0)),
        out_shape=jax.ShapeDtypeStruct((N, 40), jnp.float32),
    )(p, dp, res, b2)


def _sc_mesh():
    return plsc.VectorSubcoreMesh(core_axis_name="c", subcore_axis_name="s")


def _sc_logits(ei0, ei1, ea, eb):
    @functools.partial(
        pl.kernel,
        out_type=(
            jax.ShapeDtypeStruct((E * 16,), jnp.float32),
            jax.ShapeDtypeStruct((NWORK * 16,), jnp.float32),
        ),
        mesh=_sc_mesh(),
        scratch_types=[
            pltpu.VMEM((NCH, CH), jnp.int32),
            pltpu.VMEM((NCH, CH), jnp.int32),
            pltpu.VMEM((REM,), jnp.int32),
            pltpu.VMEM((REM,), jnp.int32),
            pltpu.VMEM((CH, 128), jnp.float32),
            pltpu.VMEM((CH, 128), jnp.float32),
            pltpu.VMEM((CH, 128), jnp.float32),
            pltpu.VMEM((CH, 128), jnp.float32),
            pltpu.VMEM((REM, 128), jnp.float32),
            pltpu.VMEM((REM, 128), jnp.float32),
            pltpu.VMEM((CH * 16,), jnp.float32),
            pltpu.VMEM((CH * 16,), jnp.float32),
            pltpu.VMEM((REM * 16,), jnp.float32),
            pltpu.VMEM((16,), jnp.float32),
            pltpu.SemaphoreType.DMA,
            pltpu.SemaphoreType.DMA,
        ],
    )
    def k(ei0_ref, ei1_ref, ea_ref, eb_ref, e_out, mx_out,
          idx2s, idx2d, sidx8, didx8, srA, drA, srB, drB, sr8, dr8,
          ebA, ebB, eb8, mxb, gA, gB):
        wid = lax.axis_index("s") * 2 + lax.axis_index("c")
        base = wid * EPT
        off8 = base + NCH * CH

        def ldids(c, _):
            off = base + c * CH
            pltpu.sync_copy(ei0_ref.at[pl.ds(off, CH)], idx2s.at[c])
            pltpu.sync_copy(ei1_ref.at[pl.ds(off, CH)], idx2d.at[c])
            return 0

        lax.fori_loop(0, NCH, ldids, 0)
        pltpu.sync_copy(ei0_ref.at[pl.ds(off8, REM)], sidx8)
        pltpu.sync_copy(ei1_ref.at[pl.ds(off8, REM)], didx8)

        def g_start(c, sr, dr, sem):
            pltpu.async_copy(ea_ref.at[idx2s.at[c]], sr, sem)
            pltpu.async_copy(eb_ref.at[idx2d.at[c]], dr, sem)

        def g_wait(c, sr, dr, sem):
            pltpu.make_async_copy(ea_ref.at[idx2s.at[c]], sr, sem).wait()
            pltpu.make_async_copy(eb_ref.at[idx2d.at[c]], dr, sem).wait()

        def compute(sr, dr, ebuf, mx):
            def ej4(q, mxq):
                for u in range(4):
                    j = q * 4 + u
                    v = sr[j, pl.ds(0, 16)] + dr[j, pl.ds(0, 16)]
                    ev = jnp.where(v > 0.0, v, NEG * v)
                    ebuf[pl.ds(j * 16, 16)] = ev
                    mxq = jnp.maximum(mxq, ev)
                return mxq

            return lax.fori_loop(0, CH // 4, ej4, mx)

        g_start(0, srA, drA, gA)

        def pair(cc, mx):
            c0 = 2 * cc
            c1 = c0 + 1
            g_start(c1, srB, drB, gB)
            g_wait(c0, srA, drA, gA)
            mx = compute(srA, drA, ebA, mx)
            pltpu.sync_copy(ebA, e_out.at[pl.ds((base + c0 * CH) * 16,
                                                CH * 16)])
            g_start(c0 + 2, srA, drA, gA)
            g_wait(c1, srB, drB, gB)
            mx = compute(srB, drB, ebB, mx)
            pltpu.sync_copy(ebB, e_out.at[pl.ds((base + c1 * CH) * 16,
                                                CH * 16)])
            return mx

        mx = lax.fori_loop(0, (NCH - 1) // 2, pair,
                           jnp.full((16,), -jnp.inf, jnp.float32))
        g_wait(NCH - 1, srA, drA, gA)
        mx = compute(srA, drA, ebA, mx)
        pltpu.sync_copy(ebA, e_out.at[pl.ds((base + (NCH - 1) * CH) * 16,
                                            CH * 16)])

        pltpu.async_copy(ea_ref.at[sidx8], sr8, gA)
        pltpu.async_copy(eb_ref.at[didx8], dr8, gB)
        pltpu.make_async_copy(ea_ref.at[sidx8], sr8, gA).wait()
        pltpu.make_async_copy(eb_ref.at[didx8], dr8, gB).wait()
        for jj in range(REM):
            v = sr8[jj, pl.ds(0, 16)] + dr8[jj, pl.ds(0, 16)]
            ev = jnp.where(v > 0.0, v, NEG * v)
            eb8[pl.ds(jj * 16, 16)] = ev
            mx = jnp.maximum(mx, ev)
        pltpu.sync_copy(eb8, e_out.at[pl.ds(off8 * 16, REM * 16)])
        mxb[...] = mx
        pltpu.sync_copy(mxb, mx_out.at[pl.ds(wid * 16, 16)])

    return k(ei0, ei1, ea, eb)


def _sc_messages(ei0, ei1, e_hbm, mxs, ftsl, nsl):
    slw = 128
    nz = slw // LANES

    @functools.partial(
        pl.kernel,
        out_type=(
            jax.ShapeDtypeStruct((2, N, 128), jnp.float32),
            jax.ShapeDtypeStruct((2, nsl, N, slw), jnp.float32),
        ),
        mesh=_sc_mesh(),
        scratch_types=[
            pltpu.VMEM((NCH, CH), jnp.int32),
            pltpu.VMEM((NCH, CH), jnp.int32),
            pltpu.VMEM((REM,), jnp.int32),
            pltpu.VMEM((REM,), jnp.int32),
            pltpu.VMEM((CH * 16,), jnp.float32),
            pltpu.VMEM((CH * 16,), jnp.float32),
            pltpu.VMEM((REM * 16,), jnp.float32),
            pltpu.VMEM((CH, slw), jnp.float32),
            pltpu.VMEM((CH, slw), jnp.float32),
            pltpu.VMEM((REM, slw), jnp.float32),
            pltpu.VMEM((NWORK * 16,), jnp.float32),
            pltpu.VMEM_SHARED((N, slw), jnp.float32),
            pltpu.SemaphoreType.DMA,
            pltpu.SemaphoreType.DMA,
            pltpu.SemaphoreType.DMA,
            pltpu.SemaphoreType.DMA,
            pltpu.SemaphoreType.DMA,
            pltpu.SemaphoreType.DMA,
            pltpu.SemaphoreType.DMA,
        ],
    )
    def k(ei0_ref, ei1_ref, e_ref, mx_ref, ft_ref, dpart, mpart,
          idx2s, idx2d, sidx8, didx8, erA, erB, er8, rowsA, rowsB, rows8,
          mxacc, macc, gsA, gsB, esA, esB, s8, ssA, ssB):
        cid = lax.axis_index("c")
        sid = lax.axis_index("s")
        wid = sid * 2 + cid
        base = wid * EPT
        slab = sid * SLAB
        off8 = base + NCH * CH
        zv = jnp.zeros((16,), jnp.float32)
        lane16 = lax.iota(jnp.int32, 16)

        def ldids(c, _):
            off = base + c * CH
            pltpu.sync_copy(ei0_ref.at[pl.ds(off, CH)], idx2s.at[c])
            pltpu.sync_copy(ei1_ref.at[pl.ds(off, CH)], idx2d.at[c])
            return 0

        lax.fori_loop(0, NCH, ldids, 0)
        pltpu.sync_copy(ei0_ref.at[pl.ds(off8, REM)], sidx8)
        pltpu.sync_copy(ei1_ref.at[pl.ds(off8, REM)], didx8)

        pltpu.sync_copy(mx_ref, mxacc)

        def mred(j, g):
            return jnp.maximum(g, mxacc[pl.ds(j * 16, 16)])

        gmax = lax.fori_loop(0, NWORK, mred,
                             jnp.full((16,), -jnp.inf, jnp.float32))
        msk = lane16 < 8

        def mkzrows(buf):
            def zr(j, _):
                for kk in range(nz):
                    buf[j, pl.ds(kk * 16, 16)] = zv
                return 0
            return zr

        zrows = mkzrows(rowsA)
        zrowsB = mkzrows(rowsB)

        def zrows8(j, _):
            for kk in range(nz):
                rows8[j, pl.ds(kk * 16, 16)] = zv
            return 0

        def zero_acc():
            for q in range(4):
                pltpu.sync_copy(rowsA, macc.at[pl.ds(slab + q * CH, CH)])
            pltpu.sync_copy(rowsA.at[pl.ds(0, SLAB - 4 * CH)],
                            macc.at[pl.ds(slab + 4 * CH, SLAB - 4 * CH)])

            @pl.when(sid == 15)
            def _():
                pltpu.sync_copy(rowsA.at[pl.ds(0, TAIL)],
                                macc.at[pl.ds(16 * SLAB, TAIL)])

        def gath_start(s, c, buf, sem):
            pltpu.async_copy(ft_ref.at[s].at[idx2s.at[c]], buf, sem)

        def gath_wait(s, c, buf, sem):
            pltpu.make_async_copy(ft_ref.at[s].at[idx2s.at[c]], buf,
                                  sem).wait()

        def eload_start(c, buf, sem):
            pltpu.async_copy(
                e_ref.at[pl.ds((base + c * CH) * 16, CH * 16)], buf, sem)

        def eload_wait(c, buf, sem):
            pltpu.make_async_copy(
                e_ref.at[pl.ds((base + c * CH) * 16, CH * 16)], buf,
                sem).wait()


        lax.fori_loop(0, CH, zrows, 0)
        lax.fori_loop(0, CH, zrowsB, 0)
        lax.fori_loop(0, REM, zrows8, 0)
        zero_acc()
        plsc.subcore_barrier()

        def fill(er_buf, rows_buf):
            def ej(j, _2):
                ee = jnp.where(msk,
                               jnp.exp(er_buf[pl.ds(j * 16, 16)] - gmax),
                               0.0)
                rows_buf[j, pl.ds(0, 16)] = ee
                return 0

            lax.fori_loop(0, CH, ej, 0)

        eload_start(0, erA, esA)

        def dpair(cc, _):
            c0 = 2 * cc
            c1 = c0 + 1

            @pl.when(cc > 0)
            def _():
                pltpu.make_async_copy(
                    rowsB, macc.at[idx2d.at[c0 - 1]], ssB).wait()

            eload_start(c1, erB, esB)
            eload_wait(c0, erA, esA)
            fill(erA, rowsA)
            pltpu.async_copy(rowsA, macc.at[idx2d.at[c0]], ssA, add=True)
            eload_wait(c1, erB, esB)
            fill(erB, rowsB)
            pltpu.async_copy(rowsB, macc.at[idx2d.at[c1]], ssB, add=True)
            pltpu.make_async_copy(rowsA, macc.at[idx2d.at[c0]], ssA).wait()
            eload_start(c0 + 2, erA, esA)
            return 0

        lax.fori_loop(0, (NCH - 1) // 2, dpair, 0)
        pltpu.make_async_copy(rowsB, macc.at[idx2d.at[NCH - 2]],
                              ssB).wait()
        eload_wait(NCH - 1, erA, esA)
        fill(erA, rowsA)
        pltpu.sync_copy(rowsA, macc.at[idx2d.at[NCH - 1]], add=True)

        pltpu.sync_copy(e_ref.at[pl.ds(off8 * 16, REM * 16)], er8)

        def ej8(j, _2):
            ee = jnp.where(msk,
                           jnp.exp(er8[pl.ds(j * 16, 16)] - gmax), 0.0)
            rows8[j, pl.ds(0, 16)] = ee
            return 0

        lax.fori_loop(0, REM, ej8, 0)
        pltpu.sync_copy(rows8, macc.at[didx8], add=True)

        plsc.subcore_barrier()
        pltpu.sync_copy(macc.at[pl.ds(slab, SLAB)],
                        dpart.at[cid, pl.ds(slab, SLAB)])

        @pl.when(sid == 15)
        def _():
            pltpu.sync_copy(macc.at[pl.ds(16 * SLAB, TAIL)],
                            dpart.at[cid, pl.ds(16 * SLAB, TAIL)])

        def slice_body(s, _s):
            h = s // 2
            hspl = jnp.broadcast_to(h, (16,))

            lax.fori_loop(0, CH, zrows, 0)
            zero_acc()
            plsc.subcore_barrier()

            gath_start(s, 0, rowsA, gsA)
            eload_start(0, erA, esA)

            def compute(rows_buf, er_buf):
                def ej4(q, _):
                    for u in range(4):
                        j = q * 4 + u
                        ev = er_buf[pl.ds(j * 16, 16)]
                        ee = jnp.exp(ev - gmax)
                        m = ee.at[hspl].get(mode="promise_in_bounds")
                        for kk in range(nz):
                            sl = pl.ds(kk * 16, 16)
                            rows_buf[j, sl] = rows_buf[j, sl] * m
                    return 0

                lax.fori_loop(0, CH // 4, ej4, 0)

            def pair(cc, _):
                c0 = 2 * cc
                c1 = c0 + 1

                @pl.when(cc > 0)
                def _():
                    pltpu.make_async_copy(
                        rowsB, macc.at[idx2d.at[c0 - 1]], ssB).wait()

                gath_start(s, c1, rowsB, gsB)
                eload_start(c1, erB, esB)
                gath_wait(s, c0, rowsA, gsA)
                eload_wait(c0, erA, esA)
                compute(rowsA, erA)
                pltpu.async_copy(rowsA, macc.at[idx2d.at[c0]], ssA,
                                 add=True)
                gath_wait(s, c1, rowsB, gsB)
                eload_wait(c1, erB, esB)
                compute(rowsB, erB)
                pltpu.async_copy(rowsB, macc.at[idx2d.at[c1]], ssB,
                                 add=True)
                pltpu.make_async_copy(rowsA, macc.at[idx2d.at[c0]],
                                      ssA).wait()
                gath_start(s, c0 + 2, rowsA, gsA)
                eload_start(c0 + 2, erA, esA)
                return 0

            npair = (NCH - 1) // 2
            lax.fori_loop(0, npair, pair, 0)
            pltpu.make_async_copy(rowsB, macc.at[idx2d.at[NCH - 2]],
                                  ssB).wait()
            gath_wait(s, NCH - 1, rowsA, gsA)
            eload_wait(NCH - 1, erA, esA)
            compute(rowsA, erA)
            pltpu.sync_copy(rowsA, macc.at[idx2d.at[NCH - 1]], add=True)

            cp8 = pltpu.async_copy(ft_ref.at[s].at[sidx8], rows8, s8)
            pltpu.sync_copy(e_ref.at[pl.ds(off8 * 16, REM * 16)], er8)
            cp8.wait()
            for jj in range(REM):
                ev = er8[pl.ds(jj * 16, 16)]
                ee = jnp.exp(ev - gmax)
                m = ee.at[hspl].get(mode="promise_in_bounds")
                for kk in range(nz):
                    sl = pl.ds(kk * 16, 16)
                    rows8[jj, sl] = rows8[jj, sl] * m
            pltpu.sync_copy(rows8, macc.at[didx8], add=True)

            plsc.subcore_barrier()
            pltpu.sync_copy(macc.at[pl.ds(slab, SLAB)],
                            mpart.at[cid, s, pl.ds(slab, SLAB)])

            @pl.when(sid == 15)
            def _():
                pltpu.sync_copy(macc.at[pl.ds(16 * SLAB, TAIL)],
                                mpart.at[cid, s, pl.ds(16 * SLAB, TAIL)])

            return 0

        lax.fori_loop(0, nsl, slice_body, 0)

    return k(ei0, ei1, e_hbm, mxs, ftsl)


def _mk_ab(al, ar, nh, d, kp):
    eye = jnp.eye(nh, dtype=jnp.float32)
    bdl = (al[:, :, None] * eye[:, None, :]).reshape(nh * d, nh)
    bdr = (ar[:, :, None] * eye[:, None, :]).reshape(nh * d, nh)
    z = jnp.zeros((kp, 8), jnp.float32)
    left = z.at[:nh * d, :nh].set(bdl)
    right = z.at[:nh * d, :nh].set(bdr)
    pad = jnp.zeros((kp, 96), jnp.float32)
    return (jnp.concatenate([left, right, pad], axis=1),
            jnp.concatenate([right, left, pad], axis=1))


def kernel(inputs, edge_index, W0, al0, ar0, b0, W1, al1, ar1, b1,
           W2, al2, ar2, b2, Wres2):
    ei0 = edge_index[0].astype(jnp.int32)
    ei1 = edge_index[1].astype(jnp.int32)
    ma0, mb0 = _mk_ab(al0, ar0, 8, HID, 2048)
    ma1, mb1 = _mk_ab(al1, ar1, 8, HID, 2048)
    ma2, mb2 = _mk_ab(al2, ar2, 1, NCLS, 128)
    w2p = jnp.zeros((2048, 128), jnp.float32).at[:, :NCLS].set(W2)

    ftsl0, ea0, eb0 = _tc0(inputs, W0, ma0, mb0)
    e0, mx0 = _sc_logits(ei0, ei1, ea0, eb0)
    dp0, mp0 = _sc_messages(ei0, ei1, e0, mx0, ftsl0, 16)

    h1, ftsl1, ea1, eb1 = _tcd0(mp0, dp0, b0.reshape(1, 2048), W1, ma1, mb1)
    e1, mx1 = _sc_logits(ei0, ei1, ea1, eb1)
    dp1, mp1 = _sc_messages(ei0, ei1, e1, mx1, ftsl1, 16)

    ftsl2, ea2, eb2, res2 = _tcd1(mp1, dp1, b1.reshape(1, 2048), h1,
                                  w2p, ma2, mb2, Wres2)
    e2, mx2 = _sc_logits(ei0, ei1, ea2, eb2)
    dp2, mp2 = _sc_messages(ei0, ei1, e2, mx2, ftsl2, 1)

    return _tcf(mp2, dp2, res2, b2.reshape(1, NCLS))


# speedup vs baseline: 1.7357x; 1.0108x over previous
import functools

import jax
import jax.numpy as jnp
from jax import lax
from jax.experimental import pallas as pl
from jax.experimental.pallas import tpu as pltpu
from jax.experimental.pallas import tpu_sc as plsc

N = 10000
E = 160000
HID = 256
NCLS = 40
NEG = 0.2
LANES = 16
NWORK = 32
EPT = E // NWORK
CH = 128
NCH = EPT // CH
REM = EPT - NCH * CH
SLAB = 624
TAIL = N - 16 * SLAB
ZR = 48
BLK = 400
HIGH = lax.Precision.HIGHEST


def _dot(a, b):
    return jnp.dot(a, b, preferred_element_type=jnp.float32)


def _elu(x):
    return jnp.where(x > 0.0, x, jnp.exp(jnp.minimum(x, 0.0)) - 1.0)


def _proj_tail(ft, ma_ref, mb_ref, ftsl_ref, ea_ref, eb_ref, nsl, slw):
    for s in range(nsl):
        ftsl_ref[s] = ft[:, s * slw:(s + 1) * slw]
    ea_ref[...] = _dot(ft, ma_ref[...])
    eb_ref[...] = _dot(ft, mb_ref[...])


def _tc0(x, w0, ma, mb):
    def body(x_ref, w_ref, ma_ref, mb_ref, ftsl_ref, ea_ref, eb_ref):
        ft = _dot(x_ref[...], w_ref[...])
        _proj_tail(ft, ma_ref, mb_ref, ftsl_ref, ea_ref, eb_ref, 16, 128)

    return pl.pallas_call(
        body,
        grid=(N // BLK,),
        in_specs=[
            pl.BlockSpec((BLK, 256), lambda i: (i, 0)),
            pl.BlockSpec((256, 2048), lambda i: (0, 0)),
            pl.BlockSpec((2048, 128), lambda i: (0, 0)),
            pl.BlockSpec((2048, 128), lambda i: (0, 0)),
        ],
        out_specs=[
            pl.BlockSpec((16, BLK, 128), lambda i: (0, i, 0)),
            pl.BlockSpec((BLK, 128), lambda i: (i, 0)),
            pl.BlockSpec((BLK, 128), lambda i: (i, 0)),
        ],
        out_shape=[
            jax.ShapeDtypeStruct((16, N, 128), jnp.float32),
            jax.ShapeDtypeStruct((N, 128), jnp.float32),
            jax.ShapeDtypeStruct((N, 128), jnp.float32),
        ],
    )(x, w0, ma, mb)


def _combine(p_ref, dp_ref, b_ref):
    u = jnp.concatenate([p_ref[0, s] + p_ref[1, s] for s in range(16)], axis=1)
    d = dp_ref[0, :, :16] + dp_ref[1, :, :16]
    cols = []
    for h in range(8):
        dh = d[:, h:h + 1]
        dsafe = jnp.where(dh > 0.0, dh, 1.0)
        cols.append(u[:, h * 256:(h + 1) * 256] / dsafe)
    return jnp.concatenate(cols, axis=1) + b_ref[...]


def _tcd0(p, dp, b, w1, ma, mb):
    def body(p_ref, dp_ref, b_ref, w_ref, ma_ref, mb_ref,
             h_ref, ftsl_ref, ea_ref, eb_ref):
        hn = _elu(_combine(p_ref, dp_ref, b_ref))
        h_ref[...] = hn
        ft = _dot(hn, w_ref[...])
        _proj_tail(ft, ma_ref, mb_ref, ftsl_ref, ea_ref, eb_ref, 16, 128)

    return pl.pallas_call(
        body,
        grid=(N // BLK,),
        in_specs=[
            pl.BlockSpec((2, 16, BLK, 128), lambda i: (0, 0, i, 0)),
            pl.BlockSpec((2, BLK, 128), lambda i: (0, i, 0)),
            pl.BlockSpec((1, 2048), lambda i: (0, 0)),
            pl.BlockSpec((2048, 2048), lambda i: (0, 0)),
            pl.BlockSpec((2048, 128), lambda i: (0, 0)),
            pl.BlockSpec((2048, 128), lambda i: (0, 0)),
        ],
        out_specs=[
            pl.BlockSpec((BLK, 2048), lambda i: (i, 0)),
            pl.BlockSpec((16, BLK, 128), lambda i: (0, i, 0)),
            pl.BlockSpec((BLK, 128), lambda i: (i, 0)),
            pl.BlockSpec((BLK, 128), lambda i: (i, 0)),
        ],
        out_shape=[
            jax.ShapeDtypeStruct((N, 2048), jnp.float32),
            jax.ShapeDtypeStruct((16, N, 128), jnp.float32),
            jax.ShapeDtypeStruct((N, 128), jnp.float32),
            jax.ShapeDtypeStruct((N, 128), jnp.float32),
        ],
    )(p, dp, b, w1, ma, mb)


def _tcd1(p, dp, b, hprev, w2p, ma2, mb2, wres2):
    def body(p_ref, dp_ref, b_ref, hp_ref, w_ref, ma_ref, mb_ref, wr_ref,
             ftsl_ref, ea_ref, eb_ref, res_ref):
        hn = _elu(_combine(p_ref, dp_ref, b_ref) + hp_ref[...])
        ft = _dot(hn, w_ref[...])
        _proj_tail(ft, ma_ref, mb_ref, ftsl_ref, ea_ref, eb_ref, 1, 128)
        res_ref[...] = _dot(hn, wr_ref[...])

    return pl.pallas_call(
        body,
        grid=(N // BLK,),
        in_specs=[
            pl.BlockSpec((2, 16, BLK, 128), lambda i: (0, 0, i, 0)),
            pl.BlockSpec((2, BLK, 128), lambda i: (0, i, 0)),
            pl.BlockSpec((1, 2048), lambda i: (0, 0)),
            pl.BlockSpec((BLK, 2048), lambda i: (i, 0)),
            pl.BlockSpec((2048, 128), lambda i: (0, 0)),
            pl.BlockSpec((128, 128), lambda i: (0, 0)),
            pl.BlockSpec((128, 128), lambda i: (0, 0)),
            pl.BlockSpec((2048, 40), lambda i: (0, 0)),
        ],
        out_specs=[
            pl.BlockSpec((1, BLK, 128), lambda i: (0, i, 0)),
            pl.BlockSpec((BLK, 128), lambda i: (i, 0)),
            pl.BlockSpec((BLK, 128), lambda i: (i, 0)),
            pl.BlockSpec((BLK, 40), lambda i: (i, 0)),
        ],
        out_shape=[
            jax.ShapeDtypeStruct((1, N, 128), jnp.float32),
            jax.ShapeDtypeStruct((N, 128), jnp.float32),
            jax.ShapeDtypeStruct((N, 128), jnp.float32),
            jax.ShapeDtypeStruct((N, 40), jnp.float32),
        ],
    )(p, dp, b, hprev, w2p, ma2, mb2, wres2)


def _tcf(p, dp, res, b2):
    def body(p_ref, dp_ref, res_ref, b_ref, o_ref):
        u = p_ref[0, 0] + p_ref[1, 0]
        d = (dp_ref[0, :, :16] + dp_ref[1, :, :16])[:, 0:1]
        dsafe = jnp.where(d > 0.0, d, 1.0)
        o_ref[...] = u[:, :NCLS] / dsafe + res_ref[...] + b_ref[...]

    return pl.pallas_call(
        body,
        grid=(N // BLK,),
        in_specs=[
            pl.BlockSpec((2, 1, BLK, 128), lambda i: (0, 0, i, 0)),
            pl.BlockSpec((2, BLK, 128), lambda i: (0, i, 0)),
            pl.BlockSpec((BLK, 40), lambda i: (i, 0)),
            pl.BlockSpec((1, 40), lambda i: (0, 0)),
        ],
        out_specs=pl.BlockSpec((BLK, 40), lambda i: (i, 0)),
        out_shape=jax.ShapeDtypeStruct((N, 40), jnp.float32),
    )(p, dp, res, b2)


def _sc_mesh():
    return plsc.VectorSubcoreMesh(core_axis_name="c", subcore_axis_name="s")


def _sc_logits(ei0, ei1, ea, eb):
    @functools.partial(
        pl.kernel,
        out_type=(
            jax.ShapeDtypeStruct((E * 16,), jnp.float32),
            jax.ShapeDtypeStruct((NWORK * 16,), jnp.float32),
        ),
        mesh=_sc_mesh(),
        scratch_types=[
            pltpu.VMEM((NCH, CH), jnp.int32),
            pltpu.VMEM((NCH, CH), jnp.int32),
            pltpu.VMEM((REM,), jnp.int32),
            pltpu.VMEM((REM,), jnp.int32),
            pltpu.VMEM((CH, 128), jnp.float32),
            pltpu.VMEM((CH, 128), jnp.float32),
            pltpu.VMEM((CH, 128), jnp.float32),
            pltpu.VMEM((CH, 128), jnp.float32),
            pltpu.VMEM((REM, 128), jnp.float32),
            pltpu.VMEM((REM, 128), jnp.float32),
            pltpu.VMEM((CH * 16,), jnp.float32),
            pltpu.VMEM((CH * 16,), jnp.float32),
            pltpu.VMEM((REM * 16,), jnp.float32),
            pltpu.VMEM((16,), jnp.float32),
            pltpu.SemaphoreType.DMA,
            pltpu.SemaphoreType.DMA,
        ],
    )
    def k(ei0_ref, ei1_ref, ea_ref, eb_ref, e_out, mx_out,
          idx2s, idx2d, sidx8, didx8, srA, drA, srB, drB, sr8, dr8,
          ebA, ebB, eb8, mxb, gA, gB):
        wid = lax.axis_index("s") * 2 + lax.axis_index("c")
        base = wid * EPT
        off8 = base + NCH * CH

        def ldids(c, _):
            off = base + c * CH
            pltpu.sync_copy(ei0_ref.at[pl.ds(off, CH)], idx2s.at[c])
            pltpu.sync_copy(ei1_ref.at[pl.ds(off, CH)], idx2d.at[c])
            return 0

        lax.fori_loop(0, NCH, ldids, 0)
        pltpu.sync_copy(ei0_ref.at[pl.ds(off8, REM)], sidx8)
        pltpu.sync_copy(ei1_ref.at[pl.ds(off8, REM)], didx8)

        def g_start(c, sr, dr, sem):
            pltpu.async_copy(ea_ref.at[idx2s.at[c]], sr, sem)
            pltpu.async_copy(eb_ref.at[idx2d.at[c]], dr, sem)

        def g_wait(c, sr, dr, sem):
            pltpu.make_async_copy(ea_ref.at[idx2s.at[c]], sr, sem).wait()
            pltpu.make_async_copy(eb_ref.at[idx2d.at[c]], dr, sem).wait()

        def compute(sr, dr, ebuf, mx):
            def ej4(q, mxq):
                for u in range(4):
                    j = q * 4 + u
                    v = sr[j, pl.ds(0, 16)] + dr[j, pl.ds(0, 16)]
                    ev = jnp.where(v > 0.0, v, NEG * v)
                    ebuf[pl.ds(j * 16, 16)] = ev
                    mxq = jnp.maximum(mxq, ev)
                return mxq

            return lax.fori_loop(0, CH // 4, ej4, mx)

        g_start(0, srA, drA, gA)

        def pair(cc, mx):
            c0 = 2 * cc
            c1 = c0 + 1
            g_start(c1, srB, drB, gB)
            g_wait(c0, srA, drA, gA)
            mx = compute(srA, drA, ebA, mx)
            pltpu.sync_copy(ebA, e_out.at[pl.ds((base + c0 * CH) * 16,
                                                CH * 16)])
            g_start(c0 + 2, srA, drA, gA)
            g_wait(c1, srB, drB, gB)
            mx = compute(srB, drB, ebB, mx)
            pltpu.sync_copy(ebB, e_out.at[pl.ds((base + c1 * CH) * 16,
                                                CH * 16)])
            return mx

        mx = lax.fori_loop(0, (NCH - 1) // 2, pair,
                           jnp.full((16,), -jnp.inf, jnp.float32))
        g_wait(NCH - 1, srA, drA, gA)
        mx = compute(srA, drA, ebA, mx)
        pltpu.sync_copy(ebA, e_out.at[pl.ds((base + (NCH - 1) * CH) * 16,
                                            CH * 16)])

        pltpu.async_copy(ea_ref.at[sidx8], sr8, gA)
        pltpu.async_copy(eb_ref.at[didx8], dr8, gB)
        pltpu.make_async_copy(ea_ref.at[sidx8], sr8, gA).wait()
        pltpu.make_async_copy(eb_ref.at[didx8], dr8, gB).wait()
        for jj in range(REM):
            v = sr8[jj, pl.ds(0, 16)] + dr8[jj, pl.ds(0, 16)]
            ev = jnp.where(v > 0.0, v, NEG * v)
            eb8[pl.ds(jj * 16, 16)] = ev
            mx = jnp.maximum(mx, ev)
        pltpu.sync_copy(eb8, e_out.at[pl.ds(off8 * 16, REM * 16)])
        mxb[...] = mx
        pltpu.sync_copy(mxb, mx_out.at[pl.ds(wid * 16, 16)])

    return k(ei0, ei1, ea, eb)


def _sc_messages(ei0, ei1, e_hbm, mxs, ftsl, nsl):
    slw = 128
    nz = slw // LANES

    @functools.partial(
        pl.kernel,
        out_type=(
            jax.ShapeDtypeStruct((2, N, 128), jnp.float32),
            jax.ShapeDtypeStruct((2, nsl, N, slw), jnp.float32),
        ),
        mesh=_sc_mesh(),
        scratch_types=[
            pltpu.VMEM((NCH, CH), jnp.int32),
            pltpu.VMEM((NCH, CH), jnp.int32),
            pltpu.VMEM((REM,), jnp.int32),
            pltpu.VMEM((REM,), jnp.int32),
            pltpu.VMEM((CH * 16,), jnp.float32),
            pltpu.VMEM((CH * 16,), jnp.float32),
            pltpu.VMEM((REM * 16,), jnp.float32),
            pltpu.VMEM((CH, slw), jnp.float32),
            pltpu.VMEM((CH, slw), jnp.float32),
            pltpu.VMEM((REM, slw), jnp.float32),
            pltpu.VMEM((NWORK * 16,), jnp.float32),
            pltpu.VMEM_SHARED((N, slw), jnp.float32),
            pltpu.SemaphoreType.DMA,
            pltpu.SemaphoreType.DMA,
            pltpu.SemaphoreType.DMA,
            pltpu.SemaphoreType.DMA,
            pltpu.SemaphoreType.DMA,
            pltpu.SemaphoreType.DMA,
            pltpu.SemaphoreType.DMA,
        ],
    )
    def k(ei0_ref, ei1_ref, e_ref, mx_ref, ft_ref, dpart, mpart,
          idx2s, idx2d, sidx8, didx8, erA, erB, er8, rowsA, rowsB, rows8,
          mxacc, macc, gsA, gsB, esA, esB, s8, ssA, ssB):
        cid = lax.axis_index("c")
        sid = lax.axis_index("s")
        wid = sid * 2 + cid
        base = wid * EPT
        slab = sid * SLAB
        off8 = base + NCH * CH
        zv = jnp.zeros((16,), jnp.float32)
        lane16 = lax.iota(jnp.int32, 16)

        def ldids(c, _):
            off = base + c * CH
            pltpu.sync_copy(ei0_ref.at[pl.ds(off, CH)], idx2s.at[c])
            pltpu.sync_copy(ei1_ref.at[pl.ds(off, CH)], idx2d.at[c])
            return 0

        lax.fori_loop(0, NCH, ldids, 0)
        pltpu.sync_copy(ei0_ref.at[pl.ds(off8, REM)], sidx8)
        pltpu.sync_copy(ei1_ref.at[pl.ds(off8, REM)], didx8)

        pltpu.sync_copy(mx_ref, mxacc)

        def mred(j, g):
            return jnp.maximum(g, mxacc[pl.ds(j * 16, 16)])

        gmax = lax.fori_loop(0, NWORK, mred,
                             jnp.full((16,), -jnp.inf, jnp.float32))
        msk = lane16 < 8

        def mkzrows(buf):
            def zr(j, _):
                for kk in range(nz):
                    buf[j, pl.ds(kk * 16, 16)] = zv
                return 0
            return zr

        zrows = mkzrows(rowsA)
        zrowsB = mkzrows(rowsB)

        def zrows8(j, _):
            for kk in range(nz):
                rows8[j, pl.ds(kk * 16, 16)] = zv
            return 0

        def zero_acc():
            for q in range(4):
                pltpu.sync_copy(rowsA, macc.at[pl.ds(slab + q * CH, CH)])
            pltpu.sync_copy(rowsA.at[pl.ds(0, SLAB - 4 * CH)],
                            macc.at[pl.ds(slab + 4 * CH, SLAB - 4 * CH)])

            @pl.when(sid == 15)
            def _():
                pltpu.sync_copy(rowsA.at[pl.ds(0, TAIL)],
                                macc.at[pl.ds(16 * SLAB, TAIL)])

        def gath_start(s, c, buf, sem):
            pltpu.async_copy(ft_ref.at[s].at[idx2s.at[c]], buf, sem)

        def gath_wait(s, c, buf, sem):
            pltpu.make_async_copy(ft_ref.at[s].at[idx2s.at[c]], buf,
                                  sem).wait()

        def eload_start(c, buf, sem):
            pltpu.async_copy(
                e_ref.at[pl.ds((base + c * CH) * 16, CH * 16)], buf, sem)

        def eload_wait(c, buf, sem):
            pltpu.make_async_copy(
                e_ref.at[pl.ds((base + c * CH) * 16, CH * 16)], buf,
                sem).wait()


        lax.fori_loop(0, CH, zrows, 0)
        lax.fori_loop(0, CH, zrowsB, 0)
        lax.fori_loop(0, REM, zrows8, 0)
        zero_acc()
        plsc.subcore_barrier()

        def fill(er_buf, rows_buf):
            def ej(j, _2):
                ee = jnp.where(msk,
                               jnp.exp(er_buf[pl.ds(j * 16, 16)] - gmax),
                               0.0)
                rows_buf[j, pl.ds(0, 16)] = ee
                return 0

            lax.fori_loop(0, CH, ej, 0)

        eload_start(0, erA, esA)

        def dpair(cc, _):
            c0 = 2 * cc
            c1 = c0 + 1

            @pl.when(cc > 0)
            def _():
                pltpu.make_async_copy(
                    rowsB, macc.at[idx2d.at[c0 - 1]], ssB).wait()

            eload_start(c1, erB, esB)
            eload_wait(c0, erA, esA)
            fill(erA, rowsA)
            pltpu.async_copy(rowsA, macc.at[idx2d.at[c0]], ssA, add=True)
            eload_wait(c1, erB, esB)
            fill(erB, rowsB)
            pltpu.async_copy(rowsB, macc.at[idx2d.at[c1]], ssB, add=True)
            pltpu.make_async_copy(rowsA, macc.at[idx2d.at[c0]], ssA).wait()
            eload_start(c0 + 2, erA, esA)
            return 0

        lax.fori_loop(0, (NCH - 1) // 2, dpair, 0)
        pltpu.make_async_copy(rowsB, macc.at[idx2d.at[NCH - 2]],
                              ssB).wait()
        eload_wait(NCH - 1, erA, esA)
        fill(erA, rowsA)
        pltpu.sync_copy(rowsA, macc.at[idx2d.at[NCH - 1]], add=True)

        pltpu.sync_copy(e_ref.at[pl.ds(off8 * 16, REM * 16)], er8)

        def ej8(j, _2):
            ee = jnp.where(msk,
                           jnp.exp(er8[pl.ds(j * 16, 16)] - gmax), 0.0)
            rows8[j, pl.ds(0, 16)] = ee
            return 0

        lax.fori_loop(0, REM, ej8, 0)
        pltpu.sync_copy(rows8, macc.at[didx8], add=True)

        plsc.subcore_barrier()
        pltpu.sync_copy(macc.at[pl.ds(slab, SLAB)],
                        dpart.at[cid, pl.ds(slab, SLAB)])

        @pl.when(sid == 15)
        def _():
            pltpu.sync_copy(macc.at[pl.ds(16 * SLAB, TAIL)],
                            dpart.at[cid, pl.ds(16 * SLAB, TAIL)])

        def slice_body(s, _s):
            h = s // 2
            hspl = jnp.broadcast_to(h, (16,))

            lax.fori_loop(0, CH, zrows, 0)
            zero_acc()
            plsc.subcore_barrier()

            gath_start(s, 0, rowsA, gsA)
            eload_start(0, erA, esA)

            def compute(rows_buf, er_buf):
                def ej4(q, _):
                    for u in range(4):
                        j = q * 4 + u
                        ev = er_buf[pl.ds(j * 16, 16)]
                        ee = jnp.exp(ev - gmax)
                        m = ee.at[hspl].get(mode="promise_in_bounds")
                        for kk in range(nz):
                            sl = pl.ds(kk * 16, 16)
                            rows_buf[j, sl] = rows_buf[j, sl] * m
                    return 0

                lax.fori_loop(0, CH // 4, ej4, 0)

            def pair(cc, _):
                c0 = 2 * cc
                c1 = c0 + 1

                @pl.when(cc > 0)
                def _():
                    pltpu.make_async_copy(
                        rowsB, macc.at[idx2d.at[c0 - 1]], ssB).wait()

                gath_start(s, c1, rowsB, gsB)
                eload_start(c1, erB, esB)
                gath_wait(s, c0, rowsA, gsA)
                eload_wait(c0, erA, esA)
                compute(rowsA, erA)
                pltpu.async_copy(rowsA, macc.at[idx2d.at[c0]], ssA,
                                 add=True)
                gath_wait(s, c1, rowsB, gsB)
                eload_wait(c1, erB, esB)
                compute(rowsB, erB)
                pltpu.async_copy(rowsB, macc.at[idx2d.at[c1]], ssB,
                                 add=True)
                pltpu.make_async_copy(rowsA, macc.at[idx2d.at[c0]],
                                      ssA).wait()
                gath_start(s, c0 + 2, rowsA, gsA)
                eload_start(c0 + 2, erA, esA)
                return 0

            npair = (NCH - 1) // 2
            lax.fori_loop(0, npair, pair, 0)
            pltpu.make_async_copy(rowsB, macc.at[idx2d.at[NCH - 2]],
                                  ssB).wait()
            gath_wait(s, NCH - 1, rowsA, gsA)
            eload_wait(NCH - 1, erA, esA)
            compute(rowsA, erA)
            pltpu.sync_copy(rowsA, macc.at[idx2d.at[NCH - 1]], add=True)

            cp8 = pltpu.async_copy(ft_ref.at[s].at[sidx8], rows8, s8)
            pltpu.sync_copy(e_ref.at[pl.ds(off8 * 16, REM * 16)], er8)
            cp8.wait()
            for jj in range(REM):
                ev = er8[pl.ds(jj * 16, 16)]
                ee = jnp.exp(ev - gmax)
                m = ee.at[hspl].get(mode="promise_in_bounds")
                for kk in range(nz):
                    sl = pl.ds(kk * 16, 16)
                    rows8[jj, sl] = rows8[jj, sl] * m
            pltpu.sync_copy(rows8, macc.at[didx8], add=True)

            plsc.subcore_barrier()
            pltpu.sync_copy(macc.at[pl.ds(slab, SLAB)],
                            mpart.at[cid, s, pl.ds(slab, SLAB)])

            @pl.when(sid == 15)
            def _():
                pltpu.sync_copy(macc.at[pl.ds(16 * SLAB, TAIL)],
                                mpart.at[cid, s, pl.ds(16 * SLAB, TAIL)])

            return 0

        lax.fori_loop(0, nsl, slice_body, 0)

    return k(ei0, ei1, e_hbm, mxs, ftsl)


def _mk_ab(al, ar, nh, d, kp):
    eye = jnp.eye(nh, dtype=jnp.float32)
    bdl = (al[:, :, None] * eye[:, None, :]).reshape(nh * d, nh)
    bdr = (ar[:, :, None] * eye[:, None, :]).reshape(nh * d, nh)
    z = jnp.zeros((kp, 8), jnp.float32)
    left = z.at[:nh * d, :nh].set(bdl)
    right = z.at[:nh * d, :nh].set(bdr)
    pad = jnp.zeros((kp, 96), jnp.float32)
    return (jnp.concatenate([left, right, pad], axis=1),
            jnp.concatenate([right, left, pad], axis=1))


def kernel(inputs, edge_index, W0, al0, ar0, b0, W1, al1, ar1, b1,
           W2, al2, ar2, b2, Wres2):
    ei0 = edge_index[0].astype(jnp.int32)
    ei1 = edge_index[1].astype(jnp.int32)
    ma0, mb0 = _mk_ab(al0, ar0, 8, HID, 2048)
    ma1, mb1 = _mk_ab(al1, ar1, 8, HID, 2048)
    ma2, mb2 = _mk_ab(al2, ar2, 1, NCLS, 128)
    w2p = jnp.zeros((2048, 128), jnp.float32).at[:, :NCLS].set(W2)

    ftsl0, ea0, eb0 = _tc0(inputs, W0, ma0, mb0)
    e0, mx0 = _sc_logits(ei0, ei1, ea0, eb0)
    dp0, mp0 = _sc_messages(ei0, ei1, e0, mx0, ftsl0, 16)

    h1, ftsl1, ea1, eb1 = _tcd0(mp0, dp0, b0.reshape(1, 2048), W1, ma1, mb1)
    e1, mx1 = _sc_logits(ei0, ei1, ea1, eb1)
    dp1, mp1 = _sc_messages(ei0, ei1, e1, mx1, ftsl1, 16)

    ftsl2, ea2, eb2, res2 = _tcd1(mp1, dp1, b1.reshape(1, 2048), h1,
                                  w2p, ma2, mb2, Wres2)
    e2, mx2 = _sc_logits(ei0, ei1, ea2, eb2)
    dp2, mp2 = _sc_messages(ei0, ei1, e2, mx2, ftsl2, 1)

    return _tcf(mp2, dp2, res2, b2.reshape(1, NCLS))
